# baseline (speedup 1.0000x reference)
import functools
import math

import jax
import jax.numpy as jnp
from jax.experimental import pallas as pl

_NUM_BASIS = 8
_R_CUT = 5.0
_C1 = math.sqrt(3.0)
_C2 = math.sqrt(15.0)
_C20 = math.sqrt(5.0) / 2.0
_PREF = math.sqrt(2.0 / _R_CUT)

_BLOCK = 6400


def _edge_kernel(v_ref, len_ref, emb_ref, attr_ref):
    v = v_ref[...]
    x = v[0:1, :]
    y = v[1:2, :]
    z = v[2:3, :]

    r = jnp.sqrt(x * x + y * y + z * z)
    len_ref[...] = r

    inv = 1.0 / jnp.maximum(r, 1e-12)
    ux = x * inv
    uy = y * inv
    uz = z * inv

    attr = jnp.concatenate(
        [
            jnp.ones_like(ux),
            _C1 * ux,
            _C1 * uy,
            _C1 * uz,
            _C2 * ux * uy,
            _C2 * uy * uz,
            _C20 * (3.0 * uz * uz - 1.0),
            _C2 * ux * uz,
            (_C2 / 2.0) * (ux * ux - uy * uy),
        ],
        axis=0,
    )
    attr_ref[...] = attr.T

    xc = r * (1.0 / _R_CUT)
    x3 = xc * xc * xc
    x6 = x3 * x3
    x7 = x6 * xc
    x8 = x7 * xc
    fc = (1.0 - 28.0 * x6 + 48.0 * x7 - 21.0 * x8) * (xc < 1.0).astype(r.dtype)

    n = (
        jax.lax.broadcasted_iota(jnp.int32, (_NUM_BASIS, r.shape[1]), 0) + 1
    ).astype(r.dtype)
    s = jnp.sin(n * ((math.pi / _R_CUT) * r))
    emb_ref[...] = ((_PREF * s / r) * fc).T


@jax.jit
def kernel(node_feature, edge_vec, edge_index):
    del node_feature, edge_index
    num_edges = edge_vec.shape[0]
    padded = ((num_edges + _BLOCK - 1) // _BLOCK) * _BLOCK
    vt = edge_vec.T
    if padded != num_edges:
        vt = jnp.pad(vt, ((0, 0), (0, padded - num_edges)), constant_values=1.0)
    grid = padded // _BLOCK

    lenT, embT, attrT = pl.pallas_call(
        _edge_kernel,
        grid=(grid,),
        in_specs=[pl.BlockSpec((3, _BLOCK), lambda i: (0, i))],
        out_specs=[
            pl.BlockSpec((1, _BLOCK), lambda i: (0, i)),
            pl.BlockSpec((_BLOCK, _NUM_BASIS), lambda i: (i, 0)),
            pl.BlockSpec((_BLOCK, 9), lambda i: (i, 0)),
        ],
        out_shape=[
            jax.ShapeDtypeStruct((1, padded), edge_vec.dtype),
            jax.ShapeDtypeStruct((padded, _NUM_BASIS), edge_vec.dtype),
            jax.ShapeDtypeStruct((padded, 9), edge_vec.dtype),
        ],
    )(vt)

    edge_length = lenT[0, :num_edges]
    edge_embedding = embT[:num_edges]
    edge_attr = attrT[:num_edges]
    return edge_length, edge_embedding, edge_attr


# speedup vs baseline: 4.2439x; 4.2439x over previous
import functools
import math

import jax
import jax.numpy as jnp
from jax.experimental import pallas as pl

_NUM_BASIS = 8
_R_CUT = 5.0
_C1 = math.sqrt(3.0)
_C2 = math.sqrt(15.0)
_C20 = math.sqrt(5.0) / 2.0
_PREF = math.sqrt(2.0 / _R_CUT)

_BLOCK = 6400


def _edge_kernel(v_ref, len_ref, emb_ref, attr_ref):
    v = v_ref[...]
    x = v[0:1, :]
    y = v[1:2, :]
    z = v[2:3, :]

    r = jnp.sqrt(x * x + y * y + z * z)
    len_ref[...] = r

    inv = 1.0 / jnp.maximum(r, 1e-12)
    ux = x * inv
    uy = y * inv
    uz = z * inv

    attr = jnp.concatenate(
        [
            jnp.ones_like(ux),
            _C1 * ux,
            _C1 * uy,
            _C1 * uz,
            _C2 * ux * uy,
            _C2 * uy * uz,
            _C20 * (3.0 * uz * uz - 1.0),
            _C2 * ux * uz,
            (_C2 / 2.0) * (ux * ux - uy * uy),
        ],
        axis=0,
    )
    attr_ref[...] = attr

    xc = r * (1.0 / _R_CUT)
    x3 = xc * xc * xc
    x6 = x3 * x3
    x7 = x6 * xc
    x8 = x7 * xc
    fc = (1.0 - 28.0 * x6 + 48.0 * x7 - 21.0 * x8) * (xc < 1.0).astype(r.dtype)

    n = (
        jax.lax.broadcasted_iota(jnp.int32, (_NUM_BASIS, r.shape[1]), 0) + 1
    ).astype(r.dtype)
    s = jnp.sin(n * ((math.pi / _R_CUT) * r))
    emb_ref[...] = (_PREF * s / r) * fc


@jax.jit
def kernel(node_feature, edge_vec, edge_index):
    del node_feature, edge_index
    num_edges = edge_vec.shape[0]
    padded = ((num_edges + _BLOCK - 1) // _BLOCK) * _BLOCK
    vt = edge_vec.T
    if padded != num_edges:
        vt = jnp.pad(vt, ((0, 0), (0, padded - num_edges)), constant_values=1.0)
    grid = padded // _BLOCK

    lenT, embT, attrT = pl.pallas_call(
        _edge_kernel,
        grid=(grid,),
        in_specs=[pl.BlockSpec((3, _BLOCK), lambda i: (0, i))],
        out_specs=[
            pl.BlockSpec((1, _BLOCK), lambda i: (0, i)),
            pl.BlockSpec((_NUM_BASIS, _BLOCK), lambda i: (0, i)),
            pl.BlockSpec((9, _BLOCK), lambda i: (0, i)),
        ],
        out_shape=[
            jax.ShapeDtypeStruct((1, padded), edge_vec.dtype),
            jax.ShapeDtypeStruct((_NUM_BASIS, padded), edge_vec.dtype),
            jax.ShapeDtypeStruct((9, padded), edge_vec.dtype),
        ],
    )(vt)

    edge_length = lenT[0, :num_edges]
    edge_embedding = embT[:, :num_edges].T
    edge_attr = attrT[:, :num_edges].T
    return edge_length, edge_embedding, edge_attr
